# baseline (speedup 1.0000x reference)
import functools

import jax
import jax.numpy as jnp
from jax import lax
from jax.experimental import pallas as pl
from jax.experimental.pallas import tpu as pltpu
from jax.experimental.pallas import tpu_sc as plsc

_NC = 2
_NS = 16
_NW = _NC * _NS
_L = 16

_C = 64
_NSLOT = 4


def _tc_table_body(z_ref, w_ref, b_ref, o_ref):
    n = z_ref.shape[0]
    zw = lax.dot_general(
        z_ref[...], w_ref[...],
        dimension_numbers=(((1,), (1,)), ((), ())),
        preferred_element_type=jnp.float32,
    )
    o_ref[pl.ds(0, n), :] = z_ref[...].astype(jnp.bfloat16)
    o_ref[pl.ds(n, n), :] = ((1e-6 - b_ref[...]) - zw).astype(jnp.bfloat16)


def _make_table(z, W, b):
    n, d = z.shape
    return pl.pallas_call(
        _tc_table_body,
        out_shape=jax.ShapeDtypeStruct((2 * n, d), jnp.bfloat16),
    )(z, W, b.reshape(1, d))


def _edge_subgroup(load_diff_chunk, lane):
    vecsum = jnp.zeros((_L,), jnp.float32)
    for e in range(_L):
        acc = None
        for k in range(128 // (2 * _L)):
            d = load_diff_chunk(e, k)
            p = d * d
            lo, hi = plsc.unpack(p, format=plsc.PackFormat.INTERLEAVED)
            acc = (lo + hi) if acc is None else (acc + lo + hi)
        s_e = jnp.sum(acc)
        vecsum = jnp.where(lane == e, lax.broadcast(s_e, (_L,)), vecsum)
    v = jnp.maximum(vecsum, 1e-30)
    i = lax.bitcast_convert_type(v, jnp.int32)
    i = 0x5F3759DF - lax.shift_right_arithmetic(i, 1)
    r = lax.bitcast_convert_type(i, jnp.float32)
    for _ in range(3):
        r = r * (1.5 - 0.5 * v * r * r)
    return jnp.exp(-(v * r))


def _sc_body(e_per_w, idxc_hbm, t_hbm, out_hbm,
             idx_v, rb_v, rt_v, out_v, sem, sem_t):
    wid = lax.axis_index("s") * _NC + lax.axis_index("c")
    n_groups = e_per_w // _C
    tail = e_per_w - n_groups * _C
    ipw = 2 * e_per_w

    pltpu.sync_copy(idxc_hbm.at[pl.ds(wid * ipw, ipw)], idx_v)

    lane = lax.iota(jnp.int32, _L)

    def issue(g):
        slot = lax.rem(g, _NSLOT)
        pltpu.async_copy(t_hbm.at[idx_v.at[pl.ds(g * 2 * _C, 2 * _C)]],
                         rb_v.at[slot], sem.at[slot])

    def wait(g):
        slot = lax.rem(g, _NSLOT)
        pltpu.make_async_copy(t_hbm.at[idx_v.at[pl.ds(0, 2 * _C)]],
                              rb_v.at[slot], sem.at[slot]).wait()

    issue(0)
    issue(1)
    issue(2)

    if tail:
        ct = pltpu.async_copy(
            t_hbm.at[idx_v.at[pl.ds(n_groups * 2 * _C, 2 * tail)]],
            rt_v, sem_t)

    def group(g, carry):
        slot = lax.rem(g, _NSLOT)

        @pl.when(g + 3 < n_groups)
        def _():
            issue(g + 3)

        wait(g)
        off = g * _C
        for s in range(_C // _L):

            def load(e, k, s=s):
                ee = s * _L + e
                return (rb_v[slot, ee, pl.ds(k * 2 * _L, 2 * _L)]
                        + rb_v[slot, _C + ee, pl.ds(k * 2 * _L, 2 * _L)])

            out_v[pl.ds(off + s * _L, _L)] = _edge_subgroup(load, lane)
        return carry

    lax.fori_loop(0, n_groups, group, 0)

    if tail:
        ct.wait()
        for s in range(tail // _L):

            def load_t(e, k, s=s):
                ee = s * _L + e
                return (rt_v[ee, pl.ds(k * 2 * _L, 2 * _L)]
                        + rt_v[tail + ee, pl.ds(k * 2 * _L, 2 * _L)])

            out_v[pl.ds(n_groups * _C + s * _L, _L)] = _edge_subgroup(
                load_t, lane)

    pltpu.sync_copy(out_v, out_hbm.at[pl.ds(wid * e_per_w, e_per_w)])


def _sc_distance(idxc, table, n_edges):
    e_per_w = n_edges // _NW
    tail = e_per_w % _C
    assert tail % _L == 0 and tail % 8 == 0 and (e_per_w // _C) >= 4
    mesh = plsc.VectorSubcoreMesh(core_axis_name="c", subcore_axis_name="s")
    k = pl.kernel(
        functools.partial(_sc_body, e_per_w),
        out_type=jax.ShapeDtypeStruct((n_edges,), jnp.float32),
        mesh=mesh,
        compiler_params=pltpu.CompilerParams(
            needs_layout_passes=False,
            use_tc_tiling_on_sc=False,
        ),
        scratch_types=[
            pltpu.VMEM((2 * e_per_w,), jnp.int32),
            pltpu.VMEM((_NSLOT, 2 * _C, 128), jnp.bfloat16),
            pltpu.VMEM((max(2 * _L, 2 * tail), 128), jnp.bfloat16),
            pltpu.VMEM((e_per_w,), jnp.float32),
            pltpu.SemaphoreType.DMA((_NSLOT,)),
            pltpu.SemaphoreType.DMA,
        ],
    )
    return k(idxc, table)


def kernel(z, edge_index, W, b):
    e = edge_index.astype(jnp.int32)
    n_edges = e.shape[1]
    n = z.shape[0]
    e_per_w = n_edges // _NW
    full = (e_per_w // _C) * _C

    e0w = e[0].reshape(_NW, e_per_w)
    e1w = e[1].reshape(_NW, e_per_w) + n
    fullc = jnp.concatenate(
        [e0w[:, :full].reshape(_NW, -1, _C),
         e1w[:, :full].reshape(_NW, -1, _C)], axis=2)
    tailc = jnp.concatenate([e0w[:, full:], e1w[:, full:]], axis=1)
    idxc = jnp.concatenate(
        [fullc.reshape(_NW, -1), tailc], axis=1).reshape(-1)

    table = _make_table(z, W, b)
    return _sc_distance(idxc, table, n_edges)


# speedup vs baseline: 1.1070x; 1.1070x over previous
import functools

import jax
import jax.numpy as jnp
from jax import lax
from jax.experimental import pallas as pl
from jax.experimental.pallas import tpu as pltpu
from jax.experimental.pallas import tpu_sc as plsc

_NC = 2
_NS = 16
_NW = _NC * _NS
_L = 16

_C = 80


def _tc_table_body(z_ref, w_ref, b_ref, o1_ref, o2_ref):
    zw = lax.dot_general(
        z_ref[...], w_ref[...],
        dimension_numbers=(((1,), (1,)), ((), ())),
        preferred_element_type=jnp.float32,
    )
    o1_ref[...] = z_ref[...].astype(jnp.bfloat16)
    o2_ref[...] = ((1e-6 - b_ref[...]) - zw).astype(jnp.bfloat16)


def _make_tables(z, W, b):
    n, d = z.shape
    return pl.pallas_call(
        _tc_table_body,
        out_shape=[
            jax.ShapeDtypeStruct((n, d), jnp.bfloat16),
            jax.ShapeDtypeStruct((n, d), jnp.bfloat16),
        ],
    )(z, W, b.reshape(1, d))


def _sc_body(e_per_w, e0_hbm, e1_hbm, z_hbm, nzw_hbm, out_hbm,
             idx0_v, idx1_v, r0_v, r1_v, out_v, sem_a, sem_b):
    wid = lax.axis_index("s") * _NC + lax.axis_index("c")
    base = wid * e_per_w

    pltpu.sync_copy(e0_hbm.at[pl.ds(base, e_per_w)], idx0_v)
    pltpu.sync_copy(e1_hbm.at[pl.ds(base, e_per_w)], idx1_v)

    n_groups = e_per_w // _C

    def issue(g):
        slot = lax.rem(g, 6)
        pltpu.async_copy(z_hbm.at[idx0_v.at[pl.ds(g * _C, _C)]],
                         r0_v.at[slot], sem_a.at[slot])
        pltpu.async_copy(nzw_hbm.at[idx1_v.at[pl.ds(g * _C, _C)]],
                         r1_v.at[slot], sem_b.at[slot])

    def wait(g):
        slot = lax.rem(g, 6)
        pltpu.make_async_copy(z_hbm.at[idx0_v.at[pl.ds(0, _C)]],
                              r0_v.at[slot], sem_a.at[slot]).wait()
        pltpu.make_async_copy(nzw_hbm.at[idx1_v.at[pl.ds(0, _C)]],
                              r1_v.at[slot], sem_b.at[slot]).wait()

    for _g in range(5):
        issue(_g)

    def group(g, carry):
        slot = lax.rem(g, 6)

        @pl.when(g + 5 < n_groups)
        def _():
            issue(g + 5)

        wait(g)
        off = g * _C
        lane = lax.iota(jnp.int32, _L)
        for s in range(_C // _L):
            vecsum = jnp.zeros((_L,), jnp.float32)
            for e in range(_L):
                ee = s * _L + e
                acc = None
                for k in range(128 // (2 * _L)):
                    d = (r0_v[slot, ee, pl.ds(k * 2 * _L, 2 * _L)]
                         + r1_v[slot, ee, pl.ds(k * 2 * _L, 2 * _L)])
                    p = d * d
                    lo, hi = plsc.unpack(
                        p, format=plsc.PackFormat.INTERLEAVED)
                    acc = (lo + hi) if acc is None else (acc + lo + hi)
                s_e = jnp.sum(acc)
                vecsum = jnp.where(lane == e, lax.broadcast(s_e, (_L,)),
                                   vecsum)
            v = jnp.maximum(vecsum, 1e-30)
            i = lax.bitcast_convert_type(v, jnp.int32)
            i = 0x5F3759DF - lax.shift_right_arithmetic(i, 1)
            r = lax.bitcast_convert_type(i, jnp.float32)
            for _ in range(3):
                r = r * (1.5 - 0.5 * v * r * r)
            out_v[pl.ds(off + s * _L, _L)] = jnp.exp(-(v * r))
        return carry

    lax.fori_loop(0, n_groups, group, 0)

    pltpu.sync_copy(out_v, out_hbm.at[pl.ds(base, e_per_w)])


def _sc_distance(e0, e1, z_bf, nzw_bf):
    n_edges = e0.shape[0]
    assert n_edges % (_NW * _C) == 0
    e_per_w = n_edges // _NW
    mesh = plsc.VectorSubcoreMesh(core_axis_name="c", subcore_axis_name="s")
    k = pl.kernel(
        functools.partial(_sc_body, e_per_w),
        out_type=jax.ShapeDtypeStruct((n_edges,), jnp.float32),
        mesh=mesh,
        compiler_params=pltpu.CompilerParams(
            needs_layout_passes=False,
            use_tc_tiling_on_sc=False,
        ),
        scratch_types=[
            pltpu.VMEM((e_per_w,), jnp.int32),
            pltpu.VMEM((e_per_w,), jnp.int32),
            pltpu.VMEM((6, _C, 128), jnp.bfloat16),
            pltpu.VMEM((6, _C, 128), jnp.bfloat16),
            pltpu.VMEM((e_per_w,), jnp.float32),
            pltpu.SemaphoreType.DMA((6,)),
            pltpu.SemaphoreType.DMA((6,)),
        ],
    )
    return k(e0, e1, z_bf, nzw_bf)


def kernel(z, edge_index, W, b):
    e = edge_index.astype(jnp.int32)
    z_bf, nzw_bf = _make_tables(z, W, b)
    return _sc_distance(e[0], e[1], z_bf, nzw_bf)


# speedup vs baseline: 1.1108x; 1.0034x over previous
import functools

import jax
import jax.numpy as jnp
from jax import lax
from jax.experimental import pallas as pl
from jax.experimental.pallas import tpu as pltpu
from jax.experimental.pallas import tpu_sc as plsc

_NC = 2
_NS = 16
_NW = _NC * _NS
_L = 16

_C = 80


def _tc_table_body(z_ref, w_ref, b_ref, o1_ref, o2_ref):
    zw = lax.dot_general(
        z_ref[...], w_ref[...],
        dimension_numbers=(((1,), (1,)), ((), ())),
        preferred_element_type=jnp.float32,
    )
    o1_ref[...] = z_ref[...].astype(jnp.bfloat16)
    o2_ref[...] = ((1e-6 - b_ref[...]) - zw).astype(jnp.bfloat16)


def _make_tables(z, W, b):
    n, d = z.shape
    return pl.pallas_call(
        _tc_table_body,
        out_shape=[
            jax.ShapeDtypeStruct((n, d), jnp.bfloat16),
            jax.ShapeDtypeStruct((n, d), jnp.bfloat16),
        ],
    )(z, W, b.reshape(1, d))


def _sc_body(e_per_w, e0_hbm, e1_hbm, z_hbm, nzw_hbm, out_hbm,
             idx0_v, idx1_v, r0_v, r1_v, out_v, sem_a, sem_b):
    wid = lax.axis_index("s") * _NC + lax.axis_index("c")
    base = wid * e_per_w

    pltpu.sync_copy(e0_hbm.at[pl.ds(base, e_per_w)], idx0_v)
    pltpu.sync_copy(e1_hbm.at[pl.ds(base, e_per_w)], idx1_v)

    n_groups = e_per_w // _C

    def issue(g):
        slot = jnp.bitwise_and(g, 3)
        pltpu.async_copy(z_hbm.at[idx0_v.at[pl.ds(g * _C, _C)]],
                         r0_v.at[slot], sem_a.at[slot])
        pltpu.async_copy(nzw_hbm.at[idx1_v.at[pl.ds(g * _C, _C)]],
                         r1_v.at[slot], sem_b.at[slot])

    def wait(g):
        slot = jnp.bitwise_and(g, 3)
        pltpu.make_async_copy(z_hbm.at[idx0_v.at[pl.ds(0, _C)]],
                              r0_v.at[slot], sem_a.at[slot]).wait()
        pltpu.make_async_copy(nzw_hbm.at[idx1_v.at[pl.ds(0, _C)]],
                              r1_v.at[slot], sem_b.at[slot]).wait()

    issue(0)
    issue(1)
    issue(2)

    def group(g, carry):
        slot = jnp.bitwise_and(g, 3)

        @pl.when(g + 3 < n_groups)
        def _():
            issue(g + 3)

        wait(g)
        off = g * _C
        lane = lax.iota(jnp.int32, _L)
        last_idx = jnp.full((_L,), _L - 1, dtype=jnp.int32)
        for s in range(_C // _L):
            vecsum = jnp.zeros((_L,), jnp.float32)
            for e in range(_L):
                ee = s * _L + e
                acc = None
                for k in range(128 // (2 * _L)):
                    d = (r0_v[slot, ee, pl.ds(k * 2 * _L, 2 * _L)]
                         + r1_v[slot, ee, pl.ds(k * 2 * _L, 2 * _L)])
                    p = d * d
                    lo, hi = plsc.unpack(
                        p, format=plsc.PackFormat.INTERLEAVED)
                    acc = (lo + hi) if acc is None else (acc + lo + hi)
                cum = plsc.cumsum(acc)
                tot = jnp.take_along_axis(
                    cum, last_idx, axis=0, mode="promise_in_bounds")
                vecsum = jnp.where(lane == e, tot, vecsum)
            v = jnp.maximum(vecsum, 1e-30)
            i = lax.bitcast_convert_type(v, jnp.int32)
            i = 0x5F3759DF - lax.shift_right_arithmetic(i, 1)
            r = lax.bitcast_convert_type(i, jnp.float32)
            for _ in range(3):
                r = r * (1.5 - 0.5 * v * r * r)
            out_v[pl.ds(off + s * _L, _L)] = jnp.exp(-(v * r))
        return carry

    lax.fori_loop(0, n_groups, group, 0)

    pltpu.sync_copy(out_v, out_hbm.at[pl.ds(base, e_per_w)])


def _sc_distance(e0, e1, z_bf, nzw_bf):
    n_edges = e0.shape[0]
    assert n_edges % (_NW * _C) == 0
    e_per_w = n_edges // _NW
    mesh = plsc.VectorSubcoreMesh(core_axis_name="c", subcore_axis_name="s")
    k = pl.kernel(
        functools.partial(_sc_body, e_per_w),
        out_type=jax.ShapeDtypeStruct((n_edges,), jnp.float32),
        mesh=mesh,
        compiler_params=pltpu.CompilerParams(
            needs_layout_passes=False,
            use_tc_tiling_on_sc=False,
        ),
        scratch_types=[
            pltpu.VMEM((e_per_w,), jnp.int32),
            pltpu.VMEM((e_per_w,), jnp.int32),
            pltpu.VMEM((4, _C, 128), jnp.bfloat16),
            pltpu.VMEM((4, _C, 128), jnp.bfloat16),
            pltpu.VMEM((e_per_w,), jnp.float32),
            pltpu.SemaphoreType.DMA((4,)),
            pltpu.SemaphoreType.DMA((4,)),
        ],
    )
    return k(e0, e1, z_bf, nzw_bf)


def kernel(z, edge_index, W, b):
    e = edge_index.astype(jnp.int32)
    z_bf, nzw_bf = _make_tables(z, W, b)
    return _sc_distance(e[0], e[1], z_bf, nzw_bf)


# speedup vs baseline: 1.1738x; 1.0567x over previous
import functools

import jax
import jax.numpy as jnp
from jax import lax
from jax.experimental import pallas as pl
from jax.experimental.pallas import tpu as pltpu
from jax.experimental.pallas import tpu_sc as plsc

_NC = 2
_NS = 16
_NW = _NC * _NS
_L = 16

_C = 80


def _tc_table_body(z_ref, w_ref, b_ref, o1_ref, o2_ref):
    zw = lax.dot_general(
        z_ref[...], w_ref[...],
        dimension_numbers=(((1,), (1,)), ((), ())),
        preferred_element_type=jnp.float32,
    )
    o1_ref[...] = z_ref[...].astype(jnp.bfloat16)
    o2_ref[...] = ((1e-6 - b_ref[...]) - zw).astype(jnp.bfloat16)


def _make_tables(z, W, b):
    n, d = z.shape
    return pl.pallas_call(
        _tc_table_body,
        out_shape=[
            jax.ShapeDtypeStruct((n, d), jnp.bfloat16),
            jax.ShapeDtypeStruct((n, d), jnp.bfloat16),
        ],
    )(z, W, b.reshape(1, d))


def _sc_body(e_per_w, e0_hbm, e1_hbm, z_hbm, nzw_hbm, out_hbm,
             idx0_v, idx1_v, r0_v, out_v, sem_a, sem_b):
    wid = lax.axis_index("s") * _NC + lax.axis_index("c")
    base = wid * e_per_w

    pltpu.sync_copy(e0_hbm.at[pl.ds(base, e_per_w)], idx0_v)
    pltpu.sync_copy(e1_hbm.at[pl.ds(base, e_per_w)], idx1_v)

    n_groups = e_per_w // _C

    def issue_a(g):
        slot = lax.rem(g, 6)
        pltpu.async_copy(nzw_hbm.at[idx1_v.at[pl.ds(g * _C, _C)]],
                         r0_v.at[slot], sem_a.at[slot])

    def wait_a(g):
        slot = lax.rem(g, 6)
        pltpu.make_async_copy(nzw_hbm.at[idx1_v.at[pl.ds(0, _C)]],
                              r0_v.at[slot], sem_a.at[slot]).wait()

    def issue_b(g):
        slot = lax.rem(g, 6)
        pltpu.async_copy(z_hbm.at[idx0_v.at[pl.ds(g * _C, _C)]],
                         r0_v.at[slot], sem_b.at[slot], add=True)

    def wait_b(g):
        slot = lax.rem(g, 6)
        pltpu.make_async_copy(z_hbm.at[idx0_v.at[pl.ds(0, _C)]],
                              r0_v.at[slot], sem_b.at[slot]).wait()

    for _g in range(5):
        issue_a(_g)
    wait_a(0)
    issue_b(0)
    wait_a(1)
    issue_b(1)

    def group(g, carry):
        slot = lax.rem(g, 6)

        @pl.when(g + 5 < n_groups)
        def _():
            issue_a(g + 5)

        @pl.when(g + 2 < n_groups)
        def _():
            wait_a(g + 2)
            issue_b(g + 2)

        wait_b(g)
        off = g * _C
        lane = lax.iota(jnp.int32, _L)
        last_idx = jnp.full((_L,), _L - 1, dtype=jnp.int32)
        for s in range(_C // _L):
            vecsum = jnp.zeros((_L,), jnp.float32)
            for e in range(_L):
                ee = s * _L + e
                acc = None
                for k in range(128 // (2 * _L)):
                    d = r0_v[slot, ee, pl.ds(k * 2 * _L, 2 * _L)]
                    p = d * d
                    lo, hi = plsc.unpack(
                        p, format=plsc.PackFormat.INTERLEAVED)
                    acc = (lo + hi) if acc is None else (acc + lo + hi)
                cum = plsc.cumsum(acc)
                tot = jnp.take_along_axis(
                    cum, last_idx, axis=0, mode="promise_in_bounds")
                vecsum = jnp.where(lane == e, tot, vecsum)
            v = jnp.maximum(vecsum, 1e-30)
            i = lax.bitcast_convert_type(v, jnp.int32)
            i = 0x5F3759DF - lax.shift_right_arithmetic(i, 1)
            r = lax.bitcast_convert_type(i, jnp.float32)
            for _ in range(3):
                r = r * (1.5 - 0.5 * v * r * r)
            out_v[pl.ds(off + s * _L, _L)] = jnp.exp(-(v * r))
        return carry

    lax.fori_loop(0, n_groups, group, 0)

    pltpu.sync_copy(out_v, out_hbm.at[pl.ds(base, e_per_w)])


def _sc_distance(e0, e1, z_bf, nzw_bf):
    n_edges = e0.shape[0]
    assert n_edges % (_NW * _C) == 0
    e_per_w = n_edges // _NW
    mesh = plsc.VectorSubcoreMesh(core_axis_name="c", subcore_axis_name="s")
    k = pl.kernel(
        functools.partial(_sc_body, e_per_w),
        out_type=jax.ShapeDtypeStruct((n_edges,), jnp.float32),
        mesh=mesh,
        compiler_params=pltpu.CompilerParams(
            needs_layout_passes=False,
            use_tc_tiling_on_sc=False,
        ),
        scratch_types=[
            pltpu.VMEM((e_per_w,), jnp.int32),
            pltpu.VMEM((e_per_w,), jnp.int32),
            pltpu.VMEM((6, _C, 128), jnp.bfloat16),
            pltpu.VMEM((e_per_w,), jnp.float32),
            pltpu.SemaphoreType.DMA((6,)),
            pltpu.SemaphoreType.DMA((6,)),
        ],
    )
    return k(e0, e1, z_bf, nzw_bf)


def kernel(z, edge_index, W, b):
    e = edge_index.astype(jnp.int32)
    z_bf, nzw_bf = _make_tables(z, W, b)
    return _sc_distance(e[0], e[1], z_bf, nzw_bf)


# speedup vs baseline: 1.1969x; 1.0197x over previous
import functools

import jax
import jax.numpy as jnp
from jax import lax
from jax.experimental import pallas as pl
from jax.experimental.pallas import tpu as pltpu
from jax.experimental.pallas import tpu_sc as plsc

_NC = 2
_NS = 16
_NW = _NC * _NS
_L = 16

_C = 80


def _tc_table_body(z_ref, w_ref, b_ref, o1_ref, o2_ref):
    zw = lax.dot_general(
        z_ref[...], w_ref[...],
        dimension_numbers=(((1,), (1,)), ((), ())),
        preferred_element_type=jnp.float32,
    )
    o1_ref[...] = z_ref[...].astype(jnp.bfloat16)
    o2_ref[...] = ((1e-6 - b_ref[...]) - zw).astype(jnp.bfloat16)


def _make_tables(z, W, b):
    n, d = z.shape
    return pl.pallas_call(
        _tc_table_body,
        out_shape=[
            jax.ShapeDtypeStruct((n, d), jnp.bfloat16),
            jax.ShapeDtypeStruct((n, d), jnp.bfloat16),
        ],
    )(z, W, b.reshape(1, d))


def _sc_body(e_per_w, e0_hbm, e1_hbm, z_hbm, nzw_hbm, out_hbm,
             idx0_v, idx1_v, r0_v, out_v, sem_a, sem_b):
    wid = lax.axis_index("s") * _NC + lax.axis_index("c")
    base = wid * e_per_w

    pltpu.sync_copy(e0_hbm.at[pl.ds(base, e_per_w)], idx0_v)
    pltpu.sync_copy(e1_hbm.at[pl.ds(base, e_per_w)], idx1_v)

    n_groups = e_per_w // _C

    def issue_a(g):
        slot = lax.rem(g, 8)
        pltpu.async_copy(nzw_hbm.at[idx1_v.at[pl.ds(g * _C, _C)]],
                         r0_v.at[slot], sem_a.at[slot])

    def wait_a(g):
        slot = lax.rem(g, 8)
        pltpu.make_async_copy(nzw_hbm.at[idx1_v.at[pl.ds(0, _C)]],
                              r0_v.at[slot], sem_a.at[slot]).wait()

    def issue_b(g):
        slot = lax.rem(g, 8)
        pltpu.async_copy(z_hbm.at[idx0_v.at[pl.ds(g * _C, _C)]],
                         r0_v.at[slot], sem_b.at[slot], add=True)

    def wait_b(g):
        slot = lax.rem(g, 8)
        pltpu.make_async_copy(z_hbm.at[idx0_v.at[pl.ds(0, _C)]],
                              r0_v.at[slot], sem_b.at[slot]).wait()

    for _g in range(7):
        issue_a(_g)
    for _g in range(3):
        wait_a(_g)
        issue_b(_g)

    def group(g, carry):
        slot = lax.rem(g, 8)

        @pl.when(g + 7 < n_groups)
        def _():
            issue_a(g + 7)

        @pl.when(g + 3 < n_groups)
        def _():
            wait_a(g + 3)
            issue_b(g + 3)

        wait_b(g)
        off = g * _C
        lane = lax.iota(jnp.int32, _L)
        last_idx = jnp.full((_L,), _L - 1, dtype=jnp.int32)
        for s in range(_C // _L):
            vecsum = jnp.zeros((_L,), jnp.float32)
            for e in range(_L):
                ee = s * _L + e
                acc = None
                for k in range(128 // (2 * _L)):
                    d = r0_v[slot, ee, pl.ds(k * 2 * _L, 2 * _L)]
                    p = d * d
                    lo, hi = plsc.unpack(
                        p, format=plsc.PackFormat.INTERLEAVED)
                    acc = (lo + hi) if acc is None else (acc + lo + hi)
                cum = plsc.cumsum(acc)
                tot = jnp.take_along_axis(
                    cum, last_idx, axis=0, mode="promise_in_bounds")
                vecsum = jnp.where(lane == e, tot, vecsum)
            v = jnp.maximum(vecsum, 1e-30)
            i = lax.bitcast_convert_type(v, jnp.int32)
            i = 0x5F3759DF - lax.shift_right_arithmetic(i, 1)
            r = lax.bitcast_convert_type(i, jnp.float32)
            for _ in range(3):
                r = r * (1.5 - 0.5 * v * r * r)
            out_v[pl.ds(off + s * _L, _L)] = jnp.exp(-(v * r))
        return carry

    lax.fori_loop(0, n_groups, group, 0)

    pltpu.sync_copy(out_v, out_hbm.at[pl.ds(base, e_per_w)])


def _sc_distance(e0, e1, z_bf, nzw_bf):
    n_edges = e0.shape[0]
    assert n_edges % (_NW * _C) == 0
    e_per_w = n_edges // _NW
    mesh = plsc.VectorSubcoreMesh(core_axis_name="c", subcore_axis_name="s")
    k = pl.kernel(
        functools.partial(_sc_body, e_per_w),
        out_type=jax.ShapeDtypeStruct((n_edges,), jnp.float32),
        mesh=mesh,
        compiler_params=pltpu.CompilerParams(
            needs_layout_passes=False,
            use_tc_tiling_on_sc=False,
        ),
        scratch_types=[
            pltpu.VMEM((e_per_w,), jnp.int32),
            pltpu.VMEM((e_per_w,), jnp.int32),
            pltpu.VMEM((8, _C, 128), jnp.bfloat16),
            pltpu.VMEM((e_per_w,), jnp.float32),
            pltpu.SemaphoreType.DMA((8,)),
            pltpu.SemaphoreType.DMA((8,)),
        ],
    )
    return k(e0, e1, z_bf, nzw_bf)


def kernel(z, edge_index, W, b):
    e = edge_index.astype(jnp.int32)
    z_bf, nzw_bf = _make_tables(z, W, b)
    return _sc_distance(e[0], e[1], z_bf, nzw_bf)
